# baseline (speedup 1.0000x reference)
import jax
import jax.numpy as jnp
from jax.experimental import pallas as pl
from jax.experimental.pallas import tpu as pltpu

_ROWS = 2048
_COLS = 1024
_BLK = 256
_N = _ROWS // _BLK


def _loss_kernel(x_ref, y_ref, o_ref, a_ref, b_ref, m_ref, acc_ref):
    i = pl.program_id(0)

    @pl.when(i == 0)
    def _init():
        acc_ref[0] = 0.0
        acc_ref[1] = 0.0
        acc_ref[2] = 0.0
        acc_ref[3] = 0.0
        acc_ref[4] = 0.0
        acc_ref[7] = 0.0

    @pl.when(i < _N)
    def _phase1():
        x = x_ref[...]
        y = y_ref[...]
        disp = 1.0 / jnp.maximum(x, 1e-6)
        mask = (x > 0.1) & (x < 100.0) & (jnp.abs(x) < jnp.inf)
        m = mask.astype(jnp.float32)
        a = disp * m
        b = y * m
        base = i * _BLK
        a_ref[pl.ds(base, _BLK), :] = a
        b_ref[pl.ds(base, _BLK), :] = b
        m_ref[pl.ds(base, _BLK), :] = m
        acc_ref[0] += jnp.sum(m)
        acc_ref[1] += jnp.sum(a)
        acc_ref[2] += jnp.sum(b)
        acc_ref[3] += jnp.sum(a * b)
        acc_ref[4] += jnp.sum(b * b)

    @pl.when(i == _N)
    def _fit():
        cnt = jnp.maximum(acc_ref[0], 1.0)
        mean_r = acc_ref[1] / cnt
        mean_p = acc_ref[2] / cnt
        mean_rp = acc_ref[3] / cnt
        mean_pp = acc_ref[4] / cnt
        covar = mean_rp - mean_r * mean_p
        var_p = mean_pp - mean_p * mean_p
        s = jnp.maximum(covar / (var_p + 1e-8), 1e-4)
        t = mean_r - s * mean_p
        acc_ref[5] = s
        acc_ref[6] = t

    @pl.when(i >= _N)
    def _phase2():
        base = (i - _N) * _BLK
        a = a_ref[pl.ds(base, _BLK), :]
        b = b_ref[pl.ds(base, _BLK), :]
        m = m_ref[pl.ds(base, _BLK), :]
        s = acc_ref[5]
        t = acc_ref[6]
        acc_ref[7] += jnp.sum(jnp.abs(a - s * b - t * m))

    @pl.when(i == 2 * _N - 1)
    def _emit():
        cnt = jnp.maximum(acc_ref[0], 1.0)
        o_ref[...] = jnp.full((1, 1), acc_ref[7] / cnt, jnp.float32)


def kernel(render_depth, prior_disp):
    x = render_depth.reshape(_ROWS, _COLS)
    y = prior_disp.reshape(_ROWS, _COLS)

    def in_map(i):
        j = jnp.minimum(i, _N - 1)
        return (j, 0)

    out = pl.pallas_call(
        _loss_kernel,
        grid=(2 * _N,),
        in_specs=[
            pl.BlockSpec((_BLK, _COLS), in_map),
            pl.BlockSpec((_BLK, _COLS), in_map),
        ],
        out_specs=pl.BlockSpec((1, 1), lambda i: (0, 0)),
        out_shape=jax.ShapeDtypeStruct((1, 1), jnp.float32),
        scratch_shapes=[
            pltpu.VMEM((_ROWS, _COLS), jnp.float32),
            pltpu.VMEM((_ROWS, _COLS), jnp.float32),
            pltpu.VMEM((_ROWS, _COLS), jnp.float32),
            pltpu.SMEM((8,), jnp.float32),
        ],
    )(x, y)
    return out.reshape(())


# speedup vs baseline: 1.0412x; 1.0412x over previous
import jax
import jax.numpy as jnp
from jax.experimental import pallas as pl
from jax.experimental.pallas import tpu as pltpu

_ROWS = 2048
_COLS = 1024
_BLK = 256
_N = _ROWS // _BLK


def _loss_kernel(x_ref, y_ref, o_ref, a_ref, b_ref, acc_ref):
    i = pl.program_id(0)

    @pl.when(i == 0)
    def _init():
        acc_ref[0] = 0.0
        acc_ref[1] = 0.0
        acc_ref[2] = 0.0
        acc_ref[3] = 0.0
        acc_ref[4] = 0.0

    x = x_ref[...]
    y = y_ref[...]
    disp = 1.0 / jnp.maximum(x, 1e-6)
    mask = (x > 0.1) & (x < 100.0)
    zero = jnp.zeros_like(x)
    a = jnp.where(mask, disp, zero)
    b = jnp.where(mask, y, zero)
    base = i * _BLK
    a_ref[pl.ds(base, _BLK), :] = a
    b_ref[pl.ds(base, _BLK), :] = b
    acc_ref[0] += jnp.sum(mask.astype(jnp.float32))
    acc_ref[1] += jnp.sum(a)
    acc_ref[2] += jnp.sum(b)
    acc_ref[3] += jnp.sum(a * b)
    acc_ref[4] += jnp.sum(b * b)

    @pl.when(i == _N - 1)
    def _finish():
        cnt = jnp.maximum(acc_ref[0], 1.0)
        mean_r = acc_ref[1] / cnt
        mean_p = acc_ref[2] / cnt
        mean_rp = acc_ref[3] / cnt
        mean_pp = acc_ref[4] / cnt
        covar = mean_rp - mean_r * mean_p
        var_p = mean_pp - mean_p * mean_p
        s = jnp.maximum(covar / (var_p + 1e-8), 1e-4)
        t = mean_r - s * mean_p
        af = a_ref[...]
        bf = b_ref[...]
        tm = jnp.where(af > 0.0, t, 0.0)
        loss = jnp.sum(jnp.abs(af - s * bf - tm))
        o_ref[...] = jnp.full((1, 1), loss / cnt, jnp.float32)


def kernel(render_depth, prior_disp):
    x = render_depth.reshape(_ROWS, _COLS)
    y = prior_disp.reshape(_ROWS, _COLS)

    out = pl.pallas_call(
        _loss_kernel,
        grid=(_N,),
        in_specs=[
            pl.BlockSpec((_BLK, _COLS), lambda i: (i, 0)),
            pl.BlockSpec((_BLK, _COLS), lambda i: (i, 0)),
        ],
        out_specs=pl.BlockSpec((1, 1), lambda i: (0, 0)),
        out_shape=jax.ShapeDtypeStruct((1, 1), jnp.float32),
        scratch_shapes=[
            pltpu.VMEM((_ROWS, _COLS), jnp.float32),
            pltpu.VMEM((_ROWS, _COLS), jnp.float32),
            pltpu.SMEM((5,), jnp.float32),
        ],
    )(x, y)
    return out.reshape(())
